# baseline (speedup 1.0000x reference)
import functools

import jax
import jax.numpy as jnp
from jax import lax
from jax.experimental import pallas as pl
from jax.experimental.pallas import tpu as pltpu
from jax.experimental.pallas import tpu_sc as plsc

_GEN = 64
_N = 1048576
_R = 8192
_C = 128
_RB = 2048
_G = _R // _RB
_NUM_ASSOC = _N + 1 - _GEN
_SCALE = float(_NUM_ASSOC - 2)
_MAX_IND = float(_NUM_ASSOC - 1)

_NC = 2
_NS = 16
_NW = _NC * _NS
_B_PER_W = _N // _NW
_K = 4096
_NCH = _B_PER_W // _K
_NB = 3
_L = 16


def _tc_precompute(w_ref, wn_ref, d_ref):
    w = w_ref[...]
    wsh = jnp.concatenate([w[1:, :], wn_ref[0:1, :]], axis=0)
    ii = lax.broadcasted_iota(jnp.int32, (_C, _C), 0)
    cc = lax.broadcasted_iota(jnp.int32, (_C, _C), 1)
    a_mask = ((ii >= cc) & (ii <= cc + 63)).astype(jnp.float32)
    b_mask = (ii <= cc - 65).astype(jnp.float32)
    d_ref[...] = (
        jnp.dot(w, a_mask, preferred_element_type=jnp.float32)
        + jnp.dot(wsh, b_mask, preferred_element_type=jnp.float32))


@functools.cache
def _make_sc_kernel():
    mesh = plsc.VectorSubcoreMesh(core_axis_name="c", subcore_axis_name="s")

    @functools.partial(
        pl.kernel,
        out_type=jax.ShapeDtypeStruct((_N,), jnp.float32),
        mesh=mesh,
        scratch_types=(
            [pltpu.VMEM((_K,), jnp.float32) for _ in range(2)]
            + [pltpu.VMEM((_K,), jnp.int32) for _ in range(_NCH)]
            + [pltpu.VMEM((_K,), jnp.float32) for _ in range(_NB)]
            + [pltpu.SemaphoreType.DMA,
               pltpu.SemaphoreType.DMA((2,)),
               pltpu.SemaphoreType.DMA((_NB,)),
               pltpu.SemaphoreType.DMA((_NB,)),
               pltpu.VMEM_SHARED((_N,), jnp.float32)]
        ),
    )
    def _sc_body(d_hbm, x_hbm, out_hbm, *refs):
        xv = list(refs[0:2])
        idx = list(refs[2:2 + _NCH])
        val = list(refs[2 + _NCH:2 + _NCH + _NB])
        sem_d, sem_x, sem_g, sem_o, dsh = refs[2 + _NCH + _NB:]
        sid = lax.axis_index("s")
        wid = sid * _NC + lax.axis_index("c")
        base = wid * _B_PER_W

        dslice = _N // _NS
        cp_d = pltpu.async_copy(d_hbm.at[pl.ds(sid * dslice, dslice)],
                                dsh.at[pl.ds(sid * dslice, dslice)], sem_d)
        cp_x = [None, None]
        cp_x[0] = pltpu.async_copy(
            x_hbm.at[pl.ds(base, _K)], xv[0], sem_x.at[0])
        for k in range(_NCH):
            b2 = k % 2
            if k + 1 < _NCH:
                cp_x[1 - b2] = pltpu.async_copy(
                    x_hbm.at[pl.ds(base + (k + 1) * _K, _K)],
                    xv[1 - b2], sem_x.at[1 - b2])
            cp_x[b2].wait()
            xv_b = xv[b2]
            idx_k = idx[k]

            @plsc.parallel_loop(0, _K, _L, unroll=8)
            def _compute_idx(i):
                p = xv_b[pl.ds(i, _L)] * _SCALE + 1.0
                p = jnp.maximum(p, 1.0)
                p = jnp.minimum(p, _MAX_IND)
                idx_k[pl.ds(i, _L)] = p.astype(jnp.int32)

        cp_d.wait()
        plsc.subcore_barrier()

        cp_g = [None] * _NCH
        cp_o = [None] * _NCH
        for k in range(_NCH):
            b = k % _NB
            if k >= 2:
                cp_g[k - 2].wait()
                cp_o[k - 2] = pltpu.async_copy(
                    val[(k - 2) % _NB],
                    out_hbm.at[pl.ds(base + (k - 2) * _K, _K)],
                    sem_o.at[(k - 2) % _NB])
            if k >= _NB:
                cp_o[k - _NB].wait()
            cp_g[k] = pltpu.async_copy(dsh.at[idx[k]], val[b], sem_g.at[b])

        for k in range(max(_NCH - 2, 0), _NCH):
            cp_g[k].wait()
            cp_o[k] = pltpu.async_copy(
                val[k % _NB], out_hbm.at[pl.ds(base + k * _K, _K)],
                sem_o.at[k % _NB])
        for k in range(max(_NCH - _NB, 0), _NCH):
            cp_o[k].wait()

    return _sc_body


def kernel(input_data, weight_vec):
    w2 = weight_vec.reshape(_R, _C)
    d2 = pl.pallas_call(
        _tc_precompute,
        grid=(_G,),
        in_specs=[
            pl.BlockSpec((_RB, _C), lambda i: (i, 0)),
            pl.BlockSpec(
                (8, _C),
                lambda i: (jnp.minimum((i + 1) * (_RB // 8), _R // 8 - 1), 0)),
        ],
        out_specs=pl.BlockSpec((_RB, _C), lambda i: (i, 0)),
        out_shape=jax.ShapeDtypeStruct((_R, _C), jnp.float32),
    )(w2, w2)
    return _make_sc_kernel()(d2.reshape(_N), input_data)
